# baseline (speedup 1.0000x reference)
import functools
import math

import jax
import jax.numpy as jnp
from jax import lax
from jax.experimental import pallas as pl
from jax.experimental.pallas import tpu as pltpu
from jax.experimental.pallas import tpu_sc as plsc

VOCAB = 1000000
D_MODEL = 64
DPAD = 128
ROWS = 4096
COLS = 200
B = ROWS * COLS
NC = 2
NS = 16
NW = NC * NS
XPW = ROWS // NW
BPW = B // NW
CHUNK = COLS
NCH = XPW
NBUF = 4
SCALE = math.sqrt(D_MODEL)

FMT_COLS = 1024
_FMT_GRID = -(-VOCAB // FMT_COLS)


def _fmt_body(src_ref, dst_ref):
    blk = src_ref[...] * SCALE
    onehot = (lax.broadcasted_iota(jnp.int32, (D_MODEL, DPAD), 0)
              == lax.broadcasted_iota(jnp.int32, (D_MODEL, DPAD), 1)
              ).astype(jnp.float32)
    dst_ref[...] = lax.dot_general(
        blk, onehot, (((0,), (0,)), ((), ())),
        precision=lax.Precision.HIGHEST,
        preferred_element_type=jnp.float32)


_lut_format = pl.pallas_call(
    _fmt_body,
    grid=(_FMT_GRID,),
    in_specs=[pl.BlockSpec((D_MODEL, FMT_COLS), lambda j: (0, j))],
    out_specs=pl.BlockSpec((FMT_COLS, DPAD), lambda j: (j, 0)),
    out_shape=jax.ShapeDtypeStruct((VOCAB, DPAD), jnp.float32),
)

_mesh = plsc.VectorSubcoreMesh(core_axis_name="c", subcore_axis_name="s")


@functools.partial(
    pl.kernel,
    mesh=_mesh,
    out_type=jax.ShapeDtypeStruct((ROWS, COLS, DPAD), jnp.float32),
    scratch_types=[pltpu.VMEM((BPW,), jnp.int32)]
    + [pltpu.VMEM((CHUNK, DPAD), jnp.float32)] * NBUF
    + [pltpu.SemaphoreType.DMA] * (2 * NBUF),
    compiler_params=pltpu.CompilerParams(use_tc_tiling_on_sc=False,
                                         needs_layout_passes=False),
)
def _embed(x_hbm, lut_hbm, out_hbm, idx_v,
           r0, r1, r2, r3, g0, g1, g2, g3, s0, s1, s2, s3):
    rows = (r0, r1, r2, r3)
    gsem = (g0, g1, g2, g3)
    ssem = (s0, s1, s2, s3)
    wid = lax.axis_index("s") * NC + lax.axis_index("c")
    xbase = wid * XPW
    pltpu.sync_copy(x_hbm.at[pl.ds(wid * BPW, BPW)], idx_v)

    def start_gather(g, b):
        pltpu.async_copy(
            lut_hbm.at[idx_v.at[pl.ds(g * CHUNK, CHUNK)]], rows[b], gsem[b])

    def wait_gather(b):
        pltpu.make_async_copy(
            lut_hbm.at[idx_v.at[pl.ds(0, CHUNK)]], rows[b], gsem[b]).wait()

    def src_block(b):
        return rows[b].at[:, pl.ds(0, D_MODEL)]

    def out_block(g):
        return out_hbm.at[xbase + g, :, pl.ds(0, D_MODEL)]

    def wait_store(b):
        pltpu.make_async_copy(src_block(b), out_block(0), ssem[b]).wait()

    for b in range(NBUF - 1):
        start_gather(b, b)

    def group(t, carry):
        for bb in range(NBUF):
            g = t * NBUF + bb
            gl = g + NBUF - 1
            bl = (bb + NBUF - 1) % NBUF

            @pl.when(gl < NCH)
            def _():
                @pl.when(gl >= NBUF)
                def _():
                    wait_store(bl)
                start_gather(gl, bl)

            wait_gather(bb)
            pltpu.async_copy(src_block(bb), out_block(g), ssem[bb])
        return carry

    lax.fori_loop(0, NCH // NBUF, group, 0)
    for b in range(NBUF):
        wait_store(b)


def kernel(x, lut):
    lut_fmt = _lut_format(lut.T)
    out = _embed(x.reshape(B), lut_fmt)
    return out[:, :, :D_MODEL]


# speedup vs baseline: 1.3880x; 1.3880x over previous
import functools
import math

import jax
import jax.numpy as jnp
from jax import lax
from jax.experimental import pallas as pl
from jax.experimental.pallas import tpu as pltpu
from jax.experimental.pallas import tpu_sc as plsc

VOCAB = 1000000
D_MODEL = 64
DPAD = 128
ROWS = 4096
COLS = 200
B = ROWS * COLS
NC = 2
NS = 16
NW = NC * NS
XPW = ROWS // NW
BPW = B // NW
CHUNK = COLS
NCH = XPW
NBUF = 4
SCALE = math.sqrt(D_MODEL)

_mesh = plsc.VectorSubcoreMesh(core_axis_name="c", subcore_axis_name="s")


@functools.partial(
    pl.kernel,
    mesh=_mesh,
    out_type=jax.ShapeDtypeStruct((ROWS, COLS, DPAD), jnp.float32),
    scratch_types=[pltpu.VMEM((BPW,), jnp.int32)]
    + [pltpu.VMEM((CHUNK, D_MODEL), jnp.float32)] * NBUF
    + [pltpu.SemaphoreType.DMA] * (2 * NBUF),
    compiler_params=pltpu.CompilerParams(use_tc_tiling_on_sc=False,
                                         needs_layout_passes=False),
)
def _embed(x_hbm, lut_hbm, out_hbm, idx_v,
           r0, r1, r2, r3, g0, g1, g2, g3, s0, s1, s2, s3):
    rows = (r0, r1, r2, r3)
    gsem = (g0, g1, g2, g3)
    ssem = (s0, s1, s2, s3)
    wid = lax.axis_index("s") * NC + lax.axis_index("c")
    xbase = wid * XPW
    pltpu.sync_copy(x_hbm.at[pl.ds(wid * BPW, BPW)], idx_v)

    def start_gather(g, b):
        pltpu.async_copy(
            lut_hbm.at[idx_v.at[pl.ds(g * CHUNK, CHUNK)]], rows[b], gsem[b])

    def wait_gather(b):
        pltpu.make_async_copy(
            lut_hbm.at[idx_v.at[pl.ds(0, CHUNK)]], rows[b], gsem[b]).wait()

    def out_block(g):
        return out_hbm.at[xbase + g, :, pl.ds(0, D_MODEL)]

    def wait_store(b):
        pltpu.make_async_copy(rows[b], out_block(0), ssem[b]).wait()

    def scale(buf):
        def body(i, c):
            r = i * 4
            for k in range(4):
                for j in range(D_MODEL // 16):
                    sl = pl.ds(j * 16, 16)
                    buf[r + k, sl] = buf[r + k, sl] * SCALE
            return c
        lax.fori_loop(0, CHUNK // 4, body, 0)

    for b in range(NBUF - 1):
        start_gather(b, b)

    def group(t, carry):
        for bb in range(NBUF):
            g = t * NBUF + bb
            gl = g + NBUF - 1
            bl = (bb + NBUF - 1) % NBUF

            @pl.when(gl < NCH)
            def _():
                @pl.when(gl >= NBUF)
                def _():
                    wait_store(bl)
                start_gather(gl, bl)

            wait_gather(bb)
            scale(rows[bb])
            pltpu.async_copy(rows[bb], out_block(g), ssem[bb])
        return carry

    lax.fori_loop(0, NCH // NBUF, group, 0)
    for b in range(NBUF):
        wait_store(b)


def kernel(x, lut):
    out = _embed(x.reshape(B), lut)
    return out[:, :, :D_MODEL]
